# Initial kernel scaffold; baseline (speedup 1.0000x reference)
import jax
import jax.numpy as jnp
from jax.experimental import pallas as pl


def kernel(x, edge_index, edge_weight, W1a, b1a, W1b, b1b, W1c, b1c, Wch, bch, W2a, b2a, W2b, b2b, W2c, b2c, gamma, beta, Wlin, blin):
    raise NotImplementedError("write your pallas kernel here")


# speedup vs baseline: 3.6418x; 3.6418x over previous
import functools
import jax
import jax.numpy as jnp
from jax.experimental import pallas as pl
from jax.experimental.pallas import tpu as pltpu

_N = 10000
_E = 160000
_TIN = 12
_T1 = 10
_T2 = 8
_H = 128
_TN = 1000
_NB = _N // _TN
_F32 = jnp.float32


def _tconv1_body(x_ref, wa_ref, wb_ref, wc_ref, ba_ref, bb_ref, bc_ref,
                 out_ref):
    for t in range(_T1):
        pa = jnp.zeros((_TN, _H), _F32)
        pb = jnp.zeros((_TN, _H), _F32)
        pc = jnp.zeros((_TN, _H), _F32)
        for k in range(3):
            xv = x_ref[:, t + k:t + k + 1]
            pa = pa + xv * wa_ref[k:k + 1, :]
            pb = pb + xv * wb_ref[k:k + 1, :]
            pc = pc + xv * wc_ref[k:k + 1, :]
        pa = pa + ba_ref[:]
        pb = pb + bb_ref[:]
        pc = pc + bc_ref[:]
        out_ref[t] = jnp.maximum(pa * jax.nn.sigmoid(pb) + pc, 0.0)


def _tconv1(x2, wa, wb, wc, ba, bb, bc):
    return pl.pallas_call(
        _tconv1_body,
        grid=(_NB,),
        in_specs=[
            pl.BlockSpec((_TN, 16), lambda i: (i, 0)),
            pl.BlockSpec((8, _H), lambda i: (0, 0)),
            pl.BlockSpec((8, _H), lambda i: (0, 0)),
            pl.BlockSpec((8, _H), lambda i: (0, 0)),
            pl.BlockSpec((1, _H), lambda i: (0, 0)),
            pl.BlockSpec((1, _H), lambda i: (0, 0)),
            pl.BlockSpec((1, _H), lambda i: (0, 0)),
        ],
        out_specs=pl.BlockSpec((_T1, _TN, _H), lambda i: (0, i, 0)),
        out_shape=jax.ShapeDtypeStruct((_T1, _N, _H), _F32),
    )(x2, wa, wb, wc, ba, bb, bc)


def _cheb_body(tx0_ref, tx1_ref, s2_ref, wch_ref, bch_ref, out_ref):
    tx0 = tx0_ref[0]
    tx1 = tx1_ref[0]
    tx2 = -2.0 * s2_ref[0] - tx0
    acc = jnp.dot(tx0, wch_ref[0], preferred_element_type=_F32)
    acc = acc + jnp.dot(tx1, wch_ref[1], preferred_element_type=_F32)
    acc = acc + jnp.dot(tx2, wch_ref[2], preferred_element_type=_F32)
    out_ref[0] = jnp.maximum(acc + bch_ref[:], 0.0)


def _cheb_combine(tx0, tx1, s2, wch, bch2):
    return pl.pallas_call(
        _cheb_body,
        grid=(_T1, _NB),
        in_specs=[
            pl.BlockSpec((1, _TN, _H), lambda t, i: (t, i, 0)),
            pl.BlockSpec((1, _TN, _H), lambda t, i: (t, i, 0)),
            pl.BlockSpec((1, _TN, _H), lambda t, i: (t, i, 0)),
            pl.BlockSpec((3, _H, _H), lambda t, i: (0, 0, 0)),
            pl.BlockSpec((1, _H), lambda t, i: (0, 0)),
        ],
        out_specs=pl.BlockSpec((1, _TN, _H), lambda t, i: (t, i, 0)),
        out_shape=jax.ShapeDtypeStruct((_T1, _N, _H), _F32),
    )(tx0, tx1, s2, wch, bch2)


def _tconv2_body(tg_ref, wa_ref, wb_ref, wc_ref, ba_ref, bb_ref, bc_ref,
                 out_ref):
    for t in range(_T2):
        pa = jnp.zeros((_TN, _H), _F32)
        pb = jnp.zeros((_TN, _H), _F32)
        pc = jnp.zeros((_TN, _H), _F32)
        for k in range(3):
            g = tg_ref[t + k]
            pa = pa + jnp.dot(g, wa_ref[k], preferred_element_type=_F32)
            pb = pb + jnp.dot(g, wb_ref[k], preferred_element_type=_F32)
            pc = pc + jnp.dot(g, wc_ref[k], preferred_element_type=_F32)
        pa = pa + ba_ref[:]
        pb = pb + bb_ref[:]
        pc = pc + bc_ref[:]
        out_ref[t] = jnp.maximum(pa * jax.nn.sigmoid(pb) + pc, 0.0)


def _tconv2(tg, wa, wb, wc, ba, bb, bc):
    return pl.pallas_call(
        _tconv2_body,
        grid=(_NB,),
        in_specs=[
            pl.BlockSpec((_T1, _TN, _H), lambda i: (0, i, 0)),
            pl.BlockSpec((3, _H, _H), lambda i: (0, 0, 0)),
            pl.BlockSpec((3, _H, _H), lambda i: (0, 0, 0)),
            pl.BlockSpec((3, _H, _H), lambda i: (0, 0, 0)),
            pl.BlockSpec((1, _H), lambda i: (0, 0)),
            pl.BlockSpec((1, _H), lambda i: (0, 0)),
            pl.BlockSpec((1, _H), lambda i: (0, 0)),
        ],
        out_specs=pl.BlockSpec((_T2, _TN, _H), lambda i: (0, i, 0)),
        out_shape=jax.ShapeDtypeStruct((_T2, _N, _H), _F32),
    )(tg, wa, wb, wc, ba, bb, bc)


def _head_body(t2_ref, gamma_ref, beta_ref, wlin_ref, blin_ref, out_ref):
    v = t2_ref[:]
    m = jnp.mean(v, axis=(0, 2), keepdims=True)
    var = jnp.mean((v - m) ** 2, axis=(0, 2), keepdims=True)
    inv = jax.lax.rsqrt(var + 1e-5)
    g = gamma_ref[:].reshape(1, _TN, 1)
    b = beta_ref[:].reshape(1, _TN, 1)
    tn = (v - m) * inv * g + b
    h = jnp.mean(jnp.maximum(tn, 0.0), axis=0)
    out_ref[:] = jnp.dot(h, wlin_ref[:], preferred_element_type=_F32) \
        + blin_ref[:]


def _head(t2, gamma2, beta2, wlin, blin2):
    return pl.pallas_call(
        _head_body,
        grid=(_NB,),
        in_specs=[
            pl.BlockSpec((_T2, _TN, _H), lambda i: (0, i, 0)),
            pl.BlockSpec((_TN, 1), lambda i: (i, 0)),
            pl.BlockSpec((_TN, 1), lambda i: (i, 0)),
            pl.BlockSpec((_H, 12), lambda i: (0, 0)),
            pl.BlockSpec((1, 12), lambda i: (0, 0)),
        ],
        out_specs=pl.BlockSpec((_TN, 12), lambda i: (i, 0)),
        out_shape=jax.ShapeDtypeStruct((_N, 12), _F32),
    )(t2, gamma2, beta2, wlin, blin2)


def kernel(x, edge_index, edge_weight, W1a, b1a, W1b, b1b, W1c, b1c, Wch, bch,
           W2a, b2a, W2b, b2b, W2c, b2c, gamma, beta, Wlin, blin):
    src = edge_index[0].astype(jnp.int32)
    dst = edge_index[1].astype(jnp.int32)
    ew = edge_weight.astype(_F32)

    deg = jax.ops.segment_sum(ew, src, num_segments=_N)
    dis = jnp.where(deg > 0, jax.lax.rsqrt(jnp.where(deg > 0, deg, 1.0)), 0.0)
    wn = dis[src] * ew * dis[dst]

    def S(u):
        outs = []
        for t in range(_T1):
            g = u[t, src, :] * wn[:, None]
            outs.append(jax.ops.segment_sum(g, dst, num_segments=_N))
        return jnp.stack(outs)

    x2 = jnp.pad(x[0, :, :, 0].T, ((0, 0), (0, 16 - _TIN)))
    pad8 = lambda w: jnp.pad(w[:, 0, 0, :].T, ((0, 5), (0, 0)))
    h1 = _tconv1(x2, pad8(W1a), pad8(W1b), pad8(W1c),
                 b1a.reshape(1, _H), b1b.reshape(1, _H), b1c.reshape(1, _H))

    tx1 = -S(h1)
    s2 = S(tx1)
    tg = _cheb_combine(h1, tx1, s2, Wch, bch.reshape(1, _H))

    taps = lambda w: jnp.transpose(w[:, :, 0, :], (2, 1, 0))
    t2 = _tconv2(tg, taps(W2a), taps(W2b), taps(W2c),
                 b2a.reshape(1, _H), b2b.reshape(1, _H), b2c.reshape(1, _H))

    return _head(t2, gamma.reshape(_N, 1), beta.reshape(_N, 1),
                 Wlin, blin.reshape(1, 12))
